# baseline (speedup 1.0000x reference)
import jax
import jax.numpy as jnp
from jax import lax
from jax.experimental import pallas as pl
from jax.experimental.pallas import tpu as pltpu
from jax.experimental.pallas import tpu_sc as plsc

_BITS = 8
_EBIT = 8
_L = 16
_NW = 32
_CH = 2
_RND = 12582912.0


def _quant_block(x_ref, o_ref):
    x = x_ref[...]
    d = jnp.where(x >= 0, jnp.clip(x, 1e-10, None), jnp.clip(x, None, -1e-10))
    m = jnp.max(jnp.abs(d), axis=1, keepdims=True)
    e = jnp.floor(jnp.log2(m))
    e = jnp.clip(e, -(2.0 ** (_EBIT - 1)), 2.0 ** (_EBIT - 1) - 1)
    i = jnp.round(d * jnp.exp2((_BITS - 2) - e))
    i = jnp.clip(i, -(2.0 ** (_BITS - 1)), 2.0 ** (_BITS - 1) - 1)
    o_ref[...] = i * jnp.exp2(e - (_BITS - 2))


def _tc_rows(x, row0, nrows):
    B, N = x.shape
    R = 256
    return pl.pallas_call(
        _quant_block,
        grid=(nrows // R,),
        in_specs=[pl.BlockSpec((R, N), lambda i: (row0 // R + i, 0))],
        out_specs=pl.BlockSpec((R, N), lambda i: (i, 0)),
        out_shape=jax.ShapeDtypeStruct((nrows, N), x.dtype),
        compiler_params=pltpu.CompilerParams(
            dimension_semantics=("parallel",),
        ),
    )(x)


def _row_quantize(src, dst, r, nvec):
    def maxbody(it, accs):
        a = list(accs)
        for k in range(8):
            sl = pl.ds(pl.multiple_of((it * 8 + k) * _L, _L), _L)
            a[k % 4] = jnp.maximum(a[k % 4], jnp.abs(src[r, sl]))
        return tuple(a)

    init = jnp.full((_L,), 1e-10, jnp.float32)
    a0, a1, a2, a3 = lax.fori_loop(0, nvec // 8, maxbody, (init,) * 4)
    m = jnp.maximum(jnp.maximum(a0, a1), jnp.maximum(a2, a3))
    idx = lax.iota(jnp.int32, _L)
    for sh in (1, 2, 4, 8):
        m = jnp.maximum(m, m[jnp.bitwise_xor(idx, sh)])
    ebits = lax.shift_right_logical(lax.bitcast_convert_type(m, jnp.int32), 23)
    e = jnp.clip(ebits - 127, -(2 ** (_EBIT - 1)), 2 ** (_EBIT - 1) - 1)
    scale = lax.bitcast_convert_type(
        lax.shift_left(((_BITS - 2) - e) + 127, 23), jnp.float32)
    iscale = lax.bitcast_convert_type(
        lax.shift_left((e - (_BITS - 2)) + 127, 23), jnp.float32)

    @pl.when(e[0] >= -26)
    def _():
        @plsc.parallel_loop(0, nvec, step=1, unroll=8)
        def _(i):
            sl = pl.ds(pl.multiple_of(i * _L, _L), _L)
            q = (src[r, sl] * scale + _RND) - _RND
            q = jnp.minimum(q, 2.0 ** (_BITS - 1) - 1)
            dst[r, sl] = q * iscale

    @pl.when(e[0] < -26)
    def _():
        @plsc.parallel_loop(0, nvec, step=1, unroll=8)
        def _(i):
            sl = pl.ds(pl.multiple_of(i * _L, _L), _L)
            v = src[r, sl]
            d = jnp.where(v >= 0, jnp.maximum(v, 1e-10),
                          jnp.minimum(v, -1e-10))
            q = (d * scale + _RND) - _RND
            q = jnp.clip(q, -(2.0 ** (_BITS - 1)), 2.0 ** (_BITS - 1) - 1)
            dst[r, sl] = q * iscale


def _make_sc_body(row0, nrows):
    def _sc_body(x_hbm, o_hbm, in0, in1, out0, out1, si0, si1, so0, so1):
        n = x_hbm.shape[1]
        nvec = n // _L
        rows_per_w = nrows // _NW
        nch = rows_per_w // _CH
        half = nch // 2
        wid = lax.axis_index("s") * 2 + lax.axis_index("c")
        base = row0 + wid * rows_per_w
        obase = wid * rows_per_w

        ins, outs = (in0, in1), (out0, out1)
        sis, sos = (si0, si1), (so0, so1)

        def in_slice(ch):
            return x_hbm.at[pl.ds(base + ch * _CH, _CH)]

        def out_slice(ch):
            return o_hbm.at[pl.ds(obase + ch * _CH, _CH)]

        pltpu.async_copy(in_slice(0), in0, si0)
        pltpu.async_copy(in_slice(1), in1, si1)

        def outer(o, _):
            for b in range(2):
                ch = o * 2 + b
                pltpu.make_async_copy(in_slice(ch), ins[b], sis[b]).wait()

                @pl.when(o > 0)
                def _():
                    pltpu.make_async_copy(
                        outs[b], out_slice(ch - 2), sos[b]).wait()

                for r in range(_CH):
                    _row_quantize(ins[b], outs[b], r, nvec)
                pltpu.async_copy(outs[b], out_slice(ch), sos[b])

                @pl.when(o + 1 < half)
                def _():
                    pltpu.async_copy(in_slice(ch + 2), ins[b], sis[b])
            return 0

        lax.fori_loop(0, half, outer, 0)
        pltpu.make_async_copy(out0, out_slice(nch - 2), so0).wait()
        pltpu.make_async_copy(out1, out_slice(nch - 1), so1).wait()

    return _sc_body


def _sc_rows(x, row0, nrows):
    B, N = x.shape
    mesh = plsc.VectorSubcoreMesh(core_axis_name="c", subcore_axis_name="s")
    f = pl.kernel(
        _make_sc_body(row0, nrows),
        out_type=jax.ShapeDtypeStruct((nrows, N), x.dtype),
        mesh=mesh,
        scratch_types=[
            pltpu.VMEM((_CH, N), jnp.float32),
            pltpu.VMEM((_CH, N), jnp.float32),
            pltpu.VMEM((_CH, N), jnp.float32),
            pltpu.VMEM((_CH, N), jnp.float32),
            pltpu.SemaphoreType.DMA,
            pltpu.SemaphoreType.DMA,
            pltpu.SemaphoreType.DMA,
            pltpu.SemaphoreType.DMA,
        ],
    )
    return f(x)


def kernel(x):
    top = _tc_rows(x, 0, 2048)
    bot = _sc_rows(x, 2048, 2048)
    return jnp.concatenate([top, bot], axis=0)


# speedup vs baseline: 1.5193x; 1.5193x over previous
import jax
import jax.numpy as jnp
from jax import lax
from jax.experimental import pallas as pl
from jax.experimental.pallas import tpu as pltpu
from jax.experimental.pallas import tpu_sc as plsc

_BITS = 8
_EBIT = 8
_L = 16
_NW = 32
_CH = 2
_RND = 12582912.0
_S = 2304
_RC = 256


def _quant_block(x_ref, o_ref):
    x = x_ref[...]
    d = jnp.where(x >= 0, jnp.clip(x, 1e-10, None), jnp.clip(x, None, -1e-10))
    m = jnp.max(jnp.abs(d), axis=1, keepdims=True)
    e = jnp.floor(jnp.log2(m))
    e = jnp.clip(e, -(2.0 ** (_EBIT - 1)), 2.0 ** (_EBIT - 1) - 1)
    i = jnp.round(d * jnp.exp2((_BITS - 2) - e))
    i = jnp.clip(i, -(2.0 ** (_BITS - 1)), 2.0 ** (_BITS - 1) - 1)
    o_ref[...] = i * jnp.exp2(e - (_BITS - 2))


def _row_quantize_codes(src, dst, r, nvec):
    def maxbody(it, accs):
        a = list(accs)
        for k in range(8):
            sl = pl.ds(pl.multiple_of((it * 8 + k) * _L, _L), _L)
            a[k % 4] = jnp.maximum(a[k % 4], jnp.abs(src[r, sl]))
        return tuple(a)

    init = jnp.full((_L,), 1e-10, jnp.float32)
    a0, a1, a2, a3 = lax.fori_loop(0, nvec // 8, maxbody, (init,) * 4)
    m = jnp.maximum(jnp.maximum(a0, a1), jnp.maximum(a2, a3))
    idx = lax.iota(jnp.int32, _L)
    for sh in (1, 2, 4, 8):
        m = jnp.maximum(m, m[jnp.bitwise_xor(idx, sh)])
    ebits = lax.shift_right_logical(lax.bitcast_convert_type(m, jnp.int32), 23)
    e = jnp.clip(ebits - 127, -(2 ** (_EBIT - 1)), 2 ** (_EBIT - 1) - 1)
    scale = lax.bitcast_convert_type(
        lax.shift_left(((_BITS - 2) - e) + 127, 23), jnp.float32)
    iscale = lax.bitcast_convert_type(
        lax.shift_left((e - (_BITS - 2)) + 127, 23), jnp.float32)

    hv = nvec // 2

    def emit(i, qa, qb):
        w = jnp.bitwise_or(
            lax.shift_right_logical(
                lax.bitcast_convert_type(qa, jnp.int32), 16),
            lax.bitcast_convert_type(qb, jnp.int32))
        dst[r, pl.ds(pl.multiple_of(i * _L, _L), _L)] = w

    @pl.when(e[0] >= -26)
    def _():
        @plsc.parallel_loop(0, hv, step=1, unroll=4)
        def _(i):
            sa = pl.ds(pl.multiple_of(i * _L, _L), _L)
            sb = pl.ds(pl.multiple_of((hv + i) * _L, _L), _L)
            top = 2.0 ** (_BITS - 1) - 1
            qa = jnp.minimum((src[r, sa] * scale + _RND) - _RND, top) * iscale
            qb = jnp.minimum((src[r, sb] * scale + _RND) - _RND, top) * iscale
            emit(i, qa, qb)

    @pl.when(e[0] < -26)
    def _():
        @plsc.parallel_loop(0, hv, step=1, unroll=4)
        def _(i):
            def q1(sl):
                v = src[r, sl]
                d = jnp.where(v >= 0, jnp.maximum(v, 1e-10),
                              jnp.minimum(v, -1e-10))
                q = (d * scale + _RND) - _RND
                q = jnp.clip(q, -(2.0 ** (_BITS - 1)), 2.0 ** (_BITS - 1) - 1)
                return q * iscale

            qa = q1(pl.ds(pl.multiple_of(i * _L, _L), _L))
            qb = q1(pl.ds(pl.multiple_of((hv + i) * _L, _L), _L))
            emit(i, qa, qb)


def _make_sc_body(row0, nrows):
    def _sc_body(x_hbm, c_hbm, in0, in1, out0, out1, si0, si1, so0, so1):
        n = x_hbm.shape[1]
        nvec = n // _L
        rows_per_w = nrows // _NW
        nch = rows_per_w // _CH
        half = nch // 2
        wid = lax.axis_index("s") * 2 + lax.axis_index("c")
        base = row0 + wid * rows_per_w
        obase = wid * rows_per_w

        ins, outs = (in0, in1), (out0, out1)
        sis, sos = (si0, si1), (so0, so1)

        def in_slice(ch):
            return x_hbm.at[pl.ds(base + ch * _CH, _CH)]

        def out_slice(ch):
            return c_hbm.at[pl.ds(obase + ch * _CH, _CH)]

        pltpu.async_copy(in_slice(0), in0, si0)
        pltpu.async_copy(in_slice(1), in1, si1)

        def outer(o, _):
            for b in range(2):
                ch = o * 2 + b
                pltpu.make_async_copy(in_slice(ch), ins[b], sis[b]).wait()

                @pl.when(o > 0)
                def _():
                    pltpu.make_async_copy(
                        outs[b], out_slice(ch - 2), sos[b]).wait()

                for r in range(_CH):
                    _row_quantize_codes(ins[b], outs[b], r, nvec)
                pltpu.async_copy(outs[b], out_slice(ch), sos[b])

                @pl.when(o + 1 < half)
                def _():
                    pltpu.async_copy(in_slice(ch + 2), ins[b], sis[b])
            return 0

        lax.fori_loop(0, half, outer, 0)
        pltpu.make_async_copy(out0, out_slice(nch - 2), so0).wait()
        pltpu.make_async_copy(out1, out_slice(nch - 1), so1).wait()

    return _sc_body


def _sc_codes(x, row0, nrows):
    B, N = x.shape
    mesh = plsc.VectorSubcoreMesh(core_axis_name="c", subcore_axis_name="s")
    f = pl.kernel(
        _make_sc_body(row0, nrows),
        out_type=jax.ShapeDtypeStruct((nrows, N // 2), jnp.int32),
        mesh=mesh,
        scratch_types=[
            pltpu.VMEM((_CH, N), jnp.float32),
            pltpu.VMEM((_CH, N), jnp.float32),
            pltpu.VMEM((_CH, N // 2), jnp.int32),
            pltpu.VMEM((_CH, N // 2), jnp.int32),
            pltpu.SemaphoreType.DMA,
            pltpu.SemaphoreType.DMA,
            pltpu.SemaphoreType.DMA,
            pltpu.SemaphoreType.DMA,
        ],
    )
    return f(x)


def _expand_block(_main_ref, c_ref, o_ref):
    u = c_ref[...]
    half = u.shape[1]
    o_ref[:, :half] = lax.bitcast_convert_type(
        lax.shift_left(u, 16), jnp.float32)
    o_ref[:, half:] = lax.bitcast_convert_type(
        jnp.bitwise_and(u, jnp.int32(-65536)), jnp.float32)


def kernel(x):
    B, N = x.shape
    if _S > 0:
        main = pl.pallas_call(
            _quant_block,
            grid=(_S // _RC,),
            in_specs=[pl.BlockSpec((_RC, N), lambda i: (i, 0))],
            out_specs=pl.BlockSpec((_RC, N), lambda i: (i, 0)),
            out_shape=jax.ShapeDtypeStruct((B, N), x.dtype),
            compiler_params=pltpu.CompilerParams(
                dimension_semantics=("parallel",),
            ),
        )(x)
    else:
        main = jnp.zeros((B, N), x.dtype)
    codes = _sc_codes(x, _S, B - _S)
    out = pl.pallas_call(
        _expand_block,
        grid=((B - _S) // _RC,),
        in_specs=[
            pl.BlockSpec(memory_space=pltpu.MemorySpace.HBM),
            pl.BlockSpec((_RC, N // 2), lambda i: (i, 0)),
        ],
        out_specs=pl.BlockSpec((_RC, N), lambda i: (_S // _RC + i, 0)),
        out_shape=jax.ShapeDtypeStruct((B, N), x.dtype),
        input_output_aliases={0: 0},
        name="expand_bf16",
        compiler_params=pltpu.CompilerParams(
            dimension_semantics=("parallel",),
        ),
    )(main, codes)
    return out


# speedup vs baseline: 1.5197x; 1.0003x over previous
import jax
import jax.numpy as jnp
from jax import lax
from jax.experimental import pallas as pl
from jax.experimental.pallas import tpu as pltpu
from jax.experimental.pallas import tpu_sc as plsc

_BITS = 8
_EBIT = 8
_L = 16
_NW = 32
_CH = 2
_RND = 12582912.0
_S = 2304
_RC = 256


def _quant_block(x_ref, o_ref):
    x = x_ref[...]
    d = jnp.where(x >= 0, jnp.clip(x, 1e-10, None), jnp.clip(x, None, -1e-10))
    m = jnp.max(jnp.abs(d), axis=1, keepdims=True)
    e = jnp.floor(jnp.log2(m))
    e = jnp.clip(e, -(2.0 ** (_EBIT - 1)), 2.0 ** (_EBIT - 1) - 1)
    i = jnp.round(d * jnp.exp2((_BITS - 2) - e))
    i = jnp.clip(i, -(2.0 ** (_BITS - 1)), 2.0 ** (_BITS - 1) - 1)
    o_ref[...] = i * jnp.exp2(e - (_BITS - 2))


def _row_quantize_codes(src, dst, r, nvec):
    def maxbody(it, accs):
        a = list(accs)
        for k in range(8):
            sl = pl.ds(pl.multiple_of((it * 8 + k) * _L, _L), _L)
            a[k % 4] = jnp.maximum(a[k % 4], jnp.abs(src[r, sl]))
        return tuple(a)

    init = jnp.full((_L,), 1e-10, jnp.float32)
    a0, a1, a2, a3 = lax.fori_loop(0, nvec // 8, maxbody, (init,) * 4)
    m = jnp.maximum(jnp.maximum(a0, a1), jnp.maximum(a2, a3))
    idx = lax.iota(jnp.int32, _L)
    for sh in (1, 2, 4, 8):
        m = jnp.maximum(m, m[jnp.bitwise_xor(idx, sh)])
    ebits = lax.shift_right_logical(lax.bitcast_convert_type(m, jnp.int32), 23)
    e = jnp.clip(ebits - 127, -(2 ** (_EBIT - 1)), 2 ** (_EBIT - 1) - 1)
    scale = lax.bitcast_convert_type(
        lax.shift_left(((_BITS - 2) - e) + 127, 23), jnp.float32)
    iscale = lax.bitcast_convert_type(
        lax.shift_left((e - (_BITS - 2)) + 127, 23), jnp.float32)

    hv = nvec // 2

    def emit(i, qa, qb):
        w = jnp.bitwise_or(
            lax.shift_right_logical(
                lax.bitcast_convert_type(qa, jnp.int32), 16),
            lax.bitcast_convert_type(qb, jnp.int32))
        dst[r, pl.ds(pl.multiple_of(i * _L, _L), _L)] = w

    @pl.when(e[0] >= -26)
    def _():
        @plsc.parallel_loop(0, hv, step=1, unroll=4)
        def _(i):
            sa = pl.ds(pl.multiple_of(i * _L, _L), _L)
            sb = pl.ds(pl.multiple_of((hv + i) * _L, _L), _L)
            top = 2.0 ** (_BITS - 1) - 1
            qa = jnp.minimum((src[r, sa] * scale + _RND) - _RND, top) * iscale
            qb = jnp.minimum((src[r, sb] * scale + _RND) - _RND, top) * iscale
            emit(i, qa, qb)

    @pl.when(e[0] < -26)
    def _():
        @plsc.parallel_loop(0, hv, step=1, unroll=4)
        def _(i):
            def q1(sl):
                v = src[r, sl]
                d = jnp.where(v >= 0, jnp.maximum(v, 1e-10),
                              jnp.minimum(v, -1e-10))
                q = (d * scale + _RND) - _RND
                q = jnp.clip(q, -(2.0 ** (_BITS - 1)), 2.0 ** (_BITS - 1) - 1)
                return q * iscale

            qa = q1(pl.ds(pl.multiple_of(i * _L, _L), _L))
            qb = q1(pl.ds(pl.multiple_of((hv + i) * _L, _L), _L))
            emit(i, qa, qb)


def _make_sc_body(row0, nrows):
    def _sc_body(x_hbm, c_hbm, in0, in1, out0, out1, si0, si1, so0, so1):
        n = x_hbm.shape[1]
        nvec = n // _L
        rows_per_w = nrows // _NW
        nch = rows_per_w // _CH
        half = nch // 2
        wid = lax.axis_index("s") * 2 + lax.axis_index("c")
        base = row0 + wid * rows_per_w
        obase = wid * rows_per_w

        ins, outs = (in0, in1), (out0, out1)
        sis, sos = (si0, si1), (so0, so1)

        def in_slice(ch):
            return x_hbm.at[pl.ds(base + ch * _CH, _CH)]

        def out_slice(ch):
            return c_hbm.at[pl.ds(obase + ch * _CH, _CH)]

        pltpu.async_copy(in_slice(0), in0, si0)
        pltpu.async_copy(in_slice(1), in1, si1)

        def outer(o, _):
            for b in range(2):
                ch = o * 2 + b
                pltpu.make_async_copy(in_slice(ch), ins[b], sis[b]).wait()

                @pl.when(o > 0)
                def _():
                    pltpu.make_async_copy(
                        outs[b], out_slice(ch - 2), sos[b]).wait()

                for r in range(_CH):
                    _row_quantize_codes(ins[b], outs[b], r, nvec)
                pltpu.async_copy(outs[b], out_slice(ch), sos[b])

                @pl.when(o + 1 < half)
                def _():
                    pltpu.async_copy(in_slice(ch + 2), ins[b], sis[b])
            return 0

        lax.fori_loop(0, half, outer, 0)
        pltpu.make_async_copy(out0, out_slice(nch - 2), so0).wait()
        pltpu.make_async_copy(out1, out_slice(nch - 1), so1).wait()

    return _sc_body


def _sc_codes(x, row0, nrows):
    B, N = x.shape
    mesh = plsc.VectorSubcoreMesh(core_axis_name="c", subcore_axis_name="s")
    f = pl.kernel(
        _make_sc_body(row0, nrows),
        out_type=jax.ShapeDtypeStruct((nrows, N // 2), jnp.int32),
        mesh=mesh,
        scratch_types=[
            pltpu.VMEM((_CH, N), jnp.float32),
            pltpu.VMEM((_CH, N), jnp.float32),
            pltpu.VMEM((_CH, N // 2), jnp.int32),
            pltpu.VMEM((_CH, N // 2), jnp.int32),
            pltpu.SemaphoreType.DMA,
            pltpu.SemaphoreType.DMA,
            pltpu.SemaphoreType.DMA,
            pltpu.SemaphoreType.DMA,
        ],
    )
    return f(x)


def _expand_block(_main_ref, c_ref, o_ref):
    u = c_ref[...]
    half = u.shape[1]
    o_ref[:, :half] = lax.bitcast_convert_type(
        lax.shift_left(u, 16), jnp.float32)
    o_ref[:, half:] = lax.bitcast_convert_type(
        jnp.bitwise_and(u, jnp.int32(-65536)), jnp.float32)


def kernel(x):
    B, N = x.shape
    codes = _sc_codes(x, _S, B - _S)
    if _S > 0:
        main = pl.pallas_call(
            _quant_block,
            grid=(_S // _RC,),
            in_specs=[pl.BlockSpec((_RC, N), lambda i: (i, 0))],
            out_specs=pl.BlockSpec((_RC, N), lambda i: (i, 0)),
            out_shape=jax.ShapeDtypeStruct((B, N), x.dtype),
            compiler_params=pltpu.CompilerParams(
                dimension_semantics=("parallel",),
            ),
        )(x)
    else:
        main = jnp.zeros((B, N), x.dtype)
    out = pl.pallas_call(
        _expand_block,
        grid=((B - _S) // _RC,),
        in_specs=[
            pl.BlockSpec(memory_space=pltpu.MemorySpace.HBM),
            pl.BlockSpec((_RC, N // 2), lambda i: (i, 0)),
        ],
        out_specs=pl.BlockSpec((_RC, N), lambda i: (_S // _RC + i, 0)),
        out_shape=jax.ShapeDtypeStruct((B, N), x.dtype),
        input_output_aliases={0: 0},
        name="expand_bf16",
        compiler_params=pltpu.CompilerParams(
            dimension_semantics=("parallel",),
        ),
    )(main, codes)
    return out


# speedup vs baseline: 1.6106x; 1.0598x over previous
import jax
import jax.numpy as jnp
from jax import lax
from jax.experimental import pallas as pl
from jax.experimental.pallas import tpu as pltpu
from jax.experimental.pallas import tpu_sc as plsc

_BITS = 8
_EBIT = 8
_L = 16
_NW = 32
_CH = 2
_RND = 12582912.0
_S = 2816
_RC = 256


def _quant_block(x_ref, o_ref):
    x = x_ref[...]
    d = jnp.where(x >= 0, jnp.clip(x, 1e-10, None), jnp.clip(x, None, -1e-10))
    m = jnp.max(jnp.abs(d), axis=1, keepdims=True)
    e = jnp.floor(jnp.log2(m))
    e = jnp.clip(e, -(2.0 ** (_EBIT - 1)), 2.0 ** (_EBIT - 1) - 1)
    i = jnp.round(d * jnp.exp2((_BITS - 2) - e))
    i = jnp.clip(i, -(2.0 ** (_BITS - 1)), 2.0 ** (_BITS - 1) - 1)
    o_ref[...] = i * jnp.exp2(e - (_BITS - 2))


def _row_quantize_codes(src, dst, r, nvec):
    def maxbody(it, accs):
        a = list(accs)
        for k in range(8):
            sl = pl.ds(pl.multiple_of((it * 8 + k) * _L, _L), _L)
            a[k % 4] = jnp.maximum(a[k % 4], jnp.abs(src[r, sl]))
        return tuple(a)

    init = jnp.full((_L,), 1e-10, jnp.float32)
    a0, a1, a2, a3 = lax.fori_loop(0, nvec // 8, maxbody, (init,) * 4)
    m = jnp.maximum(jnp.maximum(a0, a1), jnp.maximum(a2, a3))
    idx = lax.iota(jnp.int32, _L)
    for sh in (1, 2, 4, 8):
        m = jnp.maximum(m, m[jnp.bitwise_xor(idx, sh)])
    ebits = lax.shift_right_logical(lax.bitcast_convert_type(m, jnp.int32), 23)
    e = jnp.clip(ebits - 127, -(2 ** (_EBIT - 1)), 2 ** (_EBIT - 1) - 1)
    scale = lax.bitcast_convert_type(
        lax.shift_left(((_BITS - 2) - e) + 127, 23), jnp.float32)
    iscale = lax.bitcast_convert_type(
        lax.shift_left((e - (_BITS - 2)) + 127, 23), jnp.float32)

    hv = nvec // 2

    def emit(i, qa, qb):
        w = jnp.bitwise_or(
            lax.shift_right_logical(
                lax.bitcast_convert_type(qa, jnp.int32), 16),
            lax.bitcast_convert_type(qb, jnp.int32))
        dst[r, pl.ds(pl.multiple_of(i * _L, _L), _L)] = w

    @pl.when(e[0] >= -26)
    def _():
        @plsc.parallel_loop(0, hv, step=1, unroll=4)
        def _(i):
            sa = pl.ds(pl.multiple_of(i * _L, _L), _L)
            sb = pl.ds(pl.multiple_of((hv + i) * _L, _L), _L)
            top = 2.0 ** (_BITS - 1) - 1
            qa = jnp.minimum((src[r, sa] * scale + _RND) - _RND, top) * iscale
            qb = jnp.minimum((src[r, sb] * scale + _RND) - _RND, top) * iscale
            emit(i, qa, qb)

    @pl.when(e[0] < -26)
    def _():
        @plsc.parallel_loop(0, hv, step=1, unroll=4)
        def _(i):
            def q1(sl):
                v = src[r, sl]
                d = jnp.where(v >= 0, jnp.maximum(v, 1e-10),
                              jnp.minimum(v, -1e-10))
                q = (d * scale + _RND) - _RND
                q = jnp.clip(q, -(2.0 ** (_BITS - 1)), 2.0 ** (_BITS - 1) - 1)
                return q * iscale

            qa = q1(pl.ds(pl.multiple_of(i * _L, _L), _L))
            qb = q1(pl.ds(pl.multiple_of((hv + i) * _L, _L), _L))
            emit(i, qa, qb)


def _make_sc_body(row0, nrows):
    def _sc_body(x_hbm, c_hbm, in0, in1, out0, out1, si0, si1, so0, so1):
        n = x_hbm.shape[1]
        nvec = n // _L
        rows_per_w = nrows // _NW
        nch = rows_per_w // _CH
        half = nch // 2
        wid = lax.axis_index("s") * 2 + lax.axis_index("c")
        base = row0 + wid * rows_per_w
        obase = wid * rows_per_w

        ins, outs = (in0, in1), (out0, out1)
        sis, sos = (si0, si1), (so0, so1)

        def in_slice(ch):
            return x_hbm.at[pl.ds(base + ch * _CH, _CH)]

        def out_slice(ch):
            return c_hbm.at[pl.ds(obase + ch * _CH, _CH)]

        pltpu.async_copy(in_slice(0), in0, si0)
        pltpu.async_copy(in_slice(1), in1, si1)

        def outer(o, _):
            for b in range(2):
                ch = o * 2 + b
                pltpu.make_async_copy(in_slice(ch), ins[b], sis[b]).wait()

                @pl.when(o > 0)
                def _():
                    pltpu.make_async_copy(
                        outs[b], out_slice(ch - 2), sos[b]).wait()

                for r in range(_CH):
                    _row_quantize_codes(ins[b], outs[b], r, nvec)
                pltpu.async_copy(outs[b], out_slice(ch), sos[b])

                @pl.when(o + 1 < half)
                def _():
                    pltpu.async_copy(in_slice(ch + 2), ins[b], sis[b])
            return 0

        lax.fori_loop(0, half, outer, 0)
        pltpu.make_async_copy(out0, out_slice(nch - 2), so0).wait()
        pltpu.make_async_copy(out1, out_slice(nch - 1), so1).wait()

    return _sc_body


def _sc_codes(x, row0, nrows):
    B, N = x.shape
    mesh = plsc.VectorSubcoreMesh(core_axis_name="c", subcore_axis_name="s")
    f = pl.kernel(
        _make_sc_body(row0, nrows),
        out_type=jax.ShapeDtypeStruct((nrows, N // 2), jnp.int32),
        mesh=mesh,
        scratch_types=[
            pltpu.VMEM((_CH, N), jnp.float32),
            pltpu.VMEM((_CH, N), jnp.float32),
            pltpu.VMEM((_CH, N // 2), jnp.int32),
            pltpu.VMEM((_CH, N // 2), jnp.int32),
            pltpu.SemaphoreType.DMA,
            pltpu.SemaphoreType.DMA,
            pltpu.SemaphoreType.DMA,
            pltpu.SemaphoreType.DMA,
        ],
    )
    return f(x)


def _expand_block(_main_ref, c_ref, o_ref):
    u = c_ref[...]
    half = u.shape[1]
    o_ref[:, :half] = lax.bitcast_convert_type(
        lax.shift_left(u, 16), jnp.float32)
    o_ref[:, half:] = lax.bitcast_convert_type(
        jnp.bitwise_and(u, jnp.int32(-65536)), jnp.float32)


def kernel(x):
    B, N = x.shape
    codes = _sc_codes(x, _S, B - _S)
    if _S > 0:
        main = pl.pallas_call(
            _quant_block,
            grid=(_S // _RC,),
            in_specs=[pl.BlockSpec((_RC, N), lambda i: (i, 0))],
            out_specs=pl.BlockSpec((_RC, N), lambda i: (i, 0)),
            out_shape=jax.ShapeDtypeStruct((B, N), x.dtype),
            compiler_params=pltpu.CompilerParams(
                dimension_semantics=("parallel",),
            ),
        )(x)
    else:
        main = jnp.zeros((B, N), x.dtype)
    out = pl.pallas_call(
        _expand_block,
        grid=((B - _S) // _RC,),
        in_specs=[
            pl.BlockSpec(memory_space=pltpu.MemorySpace.HBM),
            pl.BlockSpec((_RC, N // 2), lambda i: (i, 0)),
        ],
        out_specs=pl.BlockSpec((_RC, N), lambda i: (_S // _RC + i, 0)),
        out_shape=jax.ShapeDtypeStruct((B, N), x.dtype),
        input_output_aliases={0: 0},
        name="expand_bf16",
        compiler_params=pltpu.CompilerParams(
            dimension_semantics=("parallel",),
        ),
    )(main, codes)
    return out


# speedup vs baseline: 2.1543x; 1.3376x over previous
import jax
import jax.numpy as jnp
from jax.experimental import pallas as pl
from jax.experimental.pallas import tpu as pltpu

_BITS = 8
_EBIT = 8


def _quant_block(x_ref, o_ref):
    x = x_ref[...]
    d = jnp.where(x >= 0, jnp.clip(x, 1e-10, None), jnp.clip(x, None, -1e-10))
    m = jnp.max(jnp.abs(d), axis=1, keepdims=True)
    e = jnp.floor(jnp.log2(m))
    e = jnp.clip(e, -(2.0 ** (_EBIT - 1)), 2.0 ** (_EBIT - 1) - 1)
    i = jnp.round(d * jnp.exp2((_BITS - 2) - e))
    i = jnp.clip(i, -(2.0 ** (_BITS - 1)), 2.0 ** (_BITS - 1) - 1)
    o_ref[...] = i * jnp.exp2(e - (_BITS - 2))


def kernel(x):
    B, N = x.shape
    R = 256
    return pl.pallas_call(
        _quant_block,
        grid=(B // R,),
        in_specs=[pl.BlockSpec((R, N), lambda i: (i, 0))],
        out_specs=pl.BlockSpec((R, N), lambda i: (i, 0)),
        out_shape=jax.ShapeDtypeStruct((B, N), x.dtype),
        compiler_params=pltpu.CompilerParams(
            dimension_semantics=("parallel",),
        ),
    )(x)
